# Initial kernel scaffold; baseline (speedup 1.0000x reference)
import jax
import jax.numpy as jnp
from jax.experimental import pallas as pl


def kernel(x, ln1_w, wq, wk, wv, wo, router_w, w_gate, w_up, w_down, ln2_w):
    raise NotImplementedError("write your pallas kernel here")


# speedup vs baseline: 1.5304x; 1.5304x over previous
import functools

import jax
import jax.numpy as jnp
from jax import lax
from jax.experimental import pallas as pl
from jax.experimental.pallas import tpu as pltpu
from jax.experimental.pallas import tpu_sc as plsc

B, S, D = 1, 2048, 768
H, KVH, HD = 12, 4, 64
E, K = 8, 2
F = 2688
EPS = 1e-06

TB = 256
NBLK = S * K // TB + E
CAP = NBLK * TB
BQ = 256
REP = H // KVH

NC, NS = 2, 16
NW = NC * NS
A_PER = S * K // NW
T_PER = S // NW


def _rms(x, w):
    return x * lax.rsqrt(jnp.mean(x * x, axis=-1, keepdims=True) + EPS) * w


def _rope_full(x, cosf, sinf, nh):
    parts = []
    for hh in range(nh):
        x1 = x[:, hh * HD:hh * HD + HD // 2]
        x2 = x[:, hh * HD + HD // 2:hh * HD + HD]
        parts.append(-x2)
        parts.append(x1)
    rot = jnp.concatenate(parts, axis=-1)
    return x * cosf + rot * sinf


def _qkv_body(x_ref, ln1_ref, wq_ref, wk_ref, wv_ref, cq_ref, sq_ref, ck_ref,
              sk_ref, q_ref, k_ref, v_ref):
    h = _rms(x_ref[...], ln1_ref[...])
    q = jnp.dot(h, wq_ref[...], preferred_element_type=jnp.float32)
    k = jnp.dot(h, wk_ref[...], preferred_element_type=jnp.float32)
    v = jnp.dot(h, wv_ref[...], preferred_element_type=jnp.float32)
    q = _rope_full(q, cq_ref[...], sq_ref[...], H)
    k = _rope_full(k, ck_ref[...], sk_ref[...], KVH)
    for hh in range(H):
        q_ref[hh] = q[:, hh * HD:(hh + 1) * HD]
    for hh in range(KVH):
        k_ref[hh] = k[:, hh * HD:(hh + 1) * HD]
        v_ref[hh] = v[:, hh * HD:(hh + 1) * HD]


def _attn_body(q_ref, k_ref, v_ref, o_ref):
    qi = pl.program_id(1)
    s = lax.dot_general(q_ref[0], k_ref[0], (((1,), (1,)), ((), ())),
                        preferred_element_type=jnp.float32) * 0.125
    row = lax.broadcasted_iota(jnp.int32, (BQ, S), 0) + qi * BQ
    col = lax.broadcasted_iota(jnp.int32, (BQ, S), 1)
    s = jnp.where(col <= row, s, jnp.float32(-1e9))
    m = jnp.max(s, axis=-1, keepdims=True)
    p = jnp.exp(s - m)
    p = p / jnp.sum(p, axis=-1, keepdims=True)
    o_ref[0] = lax.dot_general(p, v_ref[0], (((1,), (0,)), ((), ())),
                               preferred_element_type=jnp.float32)


def _router_body(x_ref, o_ref, wo_ref, ln2_ref, rw_ref, x2_ref, h2_ref,
                 slots_ref, wts_ref, be_ref):
    o2 = jnp.concatenate([o_ref[hh] for hh in range(H)], axis=1)
    x2 = x_ref[...] + jnp.dot(o2, wo_ref[...],
                              preferred_element_type=jnp.float32)
    x2_ref[...] = x2
    h2 = _rms(x2, ln2_ref[...])
    h2_ref[...] = h2
    logits = jnp.dot(h2, rw_ref[...], preferred_element_type=jnp.float32)

    eidx = lax.broadcasted_iota(jnp.int32, (S, E), 1)
    m0 = jnp.max(logits, axis=1, keepdims=True)
    i0 = jnp.min(jnp.where(logits == m0, eidx, E), axis=1, keepdims=True)
    lm = jnp.where(eidx == i0, jnp.float32(-1e30), logits)
    m1 = jnp.max(lm, axis=1, keepdims=True)
    i1 = jnp.min(jnp.where(lm == m1, eidx, E), axis=1, keepdims=True)
    w0 = 1.0 / (1.0 + jnp.exp(m1 - m0))
    w1 = 1.0 - w0

    oh0 = (eidx == i0).astype(jnp.float32)
    oh1 = (eidx == i1).astype(jnp.float32)
    c = jnp.concatenate([oh0, oh1], axis=0)
    sft = 1
    while sft < 2 * S:
        c = c + jnp.concatenate(
            [jnp.zeros((sft, E), jnp.float32), c[:-sft]], axis=0)
        sft *= 2
    cnt = c[2 * S - 1:2 * S, :]
    rank0 = jnp.sum(c[:S] * oh0, axis=1, keepdims=True) - 1.0
    rank1 = jnp.sum(c[S:] * oh1, axis=1, keepdims=True) - 1.0
    cnt_i = cnt.astype(jnp.int32)
    padded = ((cnt_i + TB - 1) >> 8) << 8
    tri = (lax.broadcasted_iota(jnp.int32, (E, E), 0)
           < lax.broadcasted_iota(jnp.int32, (E, E), 1)).astype(jnp.float32)
    poff = jnp.dot(padded.astype(jnp.float32), tri,
                   preferred_element_type=jnp.float32)
    slot0 = (jnp.sum(oh0 * poff, axis=1, keepdims=True) + rank0).astype(jnp.int32)
    slot1 = (jnp.sum(oh1 * poff, axis=1, keepdims=True) + rank1).astype(jnp.int32)
    zi = jnp.zeros((S, 6), jnp.int32)
    slots_ref[...] = jnp.concatenate([slot0, slot1, zi], axis=1)
    wts_ref[...] = jnp.concatenate(
        [w0, w1, jnp.zeros((S, 6), jnp.float32)], axis=1)

    bstart = lax.broadcasted_iota(jnp.int32, (32, E), 0) * TB
    poff_i = poff.astype(jnp.int32)
    be = jnp.sum((poff_i <= bstart).astype(jnp.int32), axis=1, keepdims=True) - 1
    be = jnp.clip(be, 0, E - 1)
    be_ref[...] = jnp.concatenate([be, jnp.zeros((32, 7), jnp.int32)], axis=1)


@functools.cache
def _make_sc_dispatch():
    mesh = plsc.VectorSubcoreMesh(core_axis_name="c", subcore_axis_name="s",
                                  num_cores=NC, num_subcores=NS)

    @functools.partial(
        pl.kernel,
        out_type=jax.ShapeDtypeStruct((CAP, D), jnp.float32),
        mesh=mesh,
        scratch_types=[
            pltpu.VMEM((A_PER,), jnp.int32),
            pltpu.VMEM((A_PER,), jnp.int32),
            pltpu.VMEM((A_PER, D), jnp.float32),
            pltpu.SemaphoreType.DMA,
            pltpu.SemaphoreType.DMA,
        ],
    )
    def dispatch(h2_hbm, slot_hbm, tok_hbm, out_hbm, tokv, slotv, rows,
                 sem1, sem2):
        wid = lax.axis_index("s") * NC + lax.axis_index("c")
        base = wid * A_PER
        pltpu.sync_copy(tok_hbm.at[pl.ds(base, A_PER)], tokv)
        pltpu.sync_copy(slot_hbm.at[pl.ds(base, A_PER)], slotv)
        pltpu.async_copy(h2_hbm.at[tokv], rows, sem1).wait()
        pltpu.async_copy(rows, out_hbm.at[slotv], sem2).wait()

    return dispatch


def _sc_dispatch(h2, slot_flat, tok_flat):
    return _make_sc_dispatch()(h2, slot_flat, tok_flat)


def _ffn_body(be_ref, x_ref, wg_ref, wu_ref, wd_ref, y_ref):
    xb = x_ref[...]
    g = jnp.dot(xb, wg_ref[0], preferred_element_type=jnp.float32)
    u = jnp.dot(xb, wu_ref[0], preferred_element_type=jnp.float32)
    a = jax.nn.silu(g) * u
    y_ref[...] = jnp.dot(a, wd_ref[0], preferred_element_type=jnp.float32)


@functools.cache
def _make_sc_combine():
    mesh = plsc.VectorSubcoreMesh(core_axis_name="c", subcore_axis_name="s",
                                  num_cores=NC, num_subcores=NS)

    @functools.partial(
        pl.kernel,
        out_type=(jax.ShapeDtypeStruct((S, D), jnp.float32),
                  jax.ShapeDtypeStruct((S, D), jnp.float32)),
        mesh=mesh,
        scratch_types=[
            pltpu.VMEM((T_PER,), jnp.int32),
            pltpu.VMEM((T_PER, D), jnp.float32),
            pltpu.SemaphoreType.DMA,
        ],
    )
    def combine(ys_hbm, pos0_hbm, pos1_hbm, y0_hbm, y1_hbm, idxv, buf, sem):
        wid = lax.axis_index("s") * NC + lax.axis_index("c")
        base = wid * T_PER
        pltpu.sync_copy(pos0_hbm.at[pl.ds(base, T_PER)], idxv)
        pltpu.async_copy(ys_hbm.at[idxv], buf, sem).wait()
        pltpu.sync_copy(buf, y0_hbm.at[pl.ds(base, T_PER)])
        pltpu.sync_copy(pos1_hbm.at[pl.ds(base, T_PER)], idxv)
        pltpu.async_copy(ys_hbm.at[idxv], buf, sem).wait()
        pltpu.sync_copy(buf, y1_hbm.at[pl.ds(base, T_PER)])

    return combine


def _sc_combine(y_sorted, slot0, slot1):
    return _make_sc_combine()(y_sorted, slot0, slot1)


def _final_body(x2_ref, wts_ref, y0_ref, y1_ref, out_ref):
    out_ref[...] = (x2_ref[...]
                    + wts_ref[:, 0:1] * y0_ref[...]
                    + wts_ref[:, 1:2] * y1_ref[...])


def kernel(x, ln1_w, wq, wk, wv, wo, router_w, w_gate, w_up, w_down, ln2_w):
    xf = x.reshape(S, D)
    ln1 = ln1_w.reshape(1, D)
    ln2 = ln2_w.reshape(1, D)

    pos = jnp.arange(S, dtype=jnp.float32)
    inv = 1.0 / (10000.0 ** (jnp.arange(0, HD, 2, dtype=jnp.float32) / HD))
    fr = pos[:, None] * inv[None, :]
    cos1 = jnp.concatenate([jnp.cos(fr), jnp.cos(fr)], axis=-1)
    sin1 = jnp.concatenate([jnp.sin(fr), jnp.sin(fr)], axis=-1)
    cq = jnp.concatenate([cos1] * H, axis=-1)
    sq = jnp.concatenate([sin1] * H, axis=-1)
    ck = jnp.concatenate([cos1] * KVH, axis=-1)
    sk = jnp.concatenate([sin1] * KVH, axis=-1)

    nsb = S // BQ
    q, k, v = pl.pallas_call(
        _qkv_body,
        grid=(nsb,),
        in_specs=[
            pl.BlockSpec((BQ, D), lambda i: (i, 0)),
            pl.BlockSpec((1, D), lambda i: (0, 0)),
            pl.BlockSpec((D, H * HD), lambda i: (0, 0)),
            pl.BlockSpec((D, KVH * HD), lambda i: (0, 0)),
            pl.BlockSpec((D, KVH * HD), lambda i: (0, 0)),
            pl.BlockSpec((BQ, H * HD), lambda i: (i, 0)),
            pl.BlockSpec((BQ, H * HD), lambda i: (i, 0)),
            pl.BlockSpec((BQ, KVH * HD), lambda i: (i, 0)),
            pl.BlockSpec((BQ, KVH * HD), lambda i: (i, 0)),
        ],
        out_specs=[
            pl.BlockSpec((H, BQ, HD), lambda i: (0, i, 0)),
            pl.BlockSpec((KVH, BQ, HD), lambda i: (0, i, 0)),
            pl.BlockSpec((KVH, BQ, HD), lambda i: (0, i, 0)),
        ],
        out_shape=[
            jax.ShapeDtypeStruct((H, S, HD), jnp.float32),
            jax.ShapeDtypeStruct((KVH, S, HD), jnp.float32),
            jax.ShapeDtypeStruct((KVH, S, HD), jnp.float32),
        ],
    )(xf, ln1, wq, wk, wv, cq, sq, ck, sk)

    o = pl.pallas_call(
        _attn_body,
        grid=(H, nsb),
        in_specs=[
            pl.BlockSpec((1, BQ, HD), lambda h, i: (h, i, 0)),
            pl.BlockSpec((1, S, HD), lambda h, i: (h // REP, 0, 0)),
            pl.BlockSpec((1, S, HD), lambda h, i: (h // REP, 0, 0)),
        ],
        out_specs=pl.BlockSpec((1, BQ, HD), lambda h, i: (h, i, 0)),
        out_shape=jax.ShapeDtypeStruct((H, S, HD), jnp.float32),
    )(q, k, v)

    x2, h2, slots, wts, be_arr = pl.pallas_call(
        _router_body,
        out_shape=[
            jax.ShapeDtypeStruct((S, D), jnp.float32),
            jax.ShapeDtypeStruct((S, D), jnp.float32),
            jax.ShapeDtypeStruct((S, E), jnp.int32),
            jax.ShapeDtypeStruct((S, E), jnp.float32),
            jax.ShapeDtypeStruct((32, E), jnp.int32),
        ],
    )(xf, o, wo, ln2, router_w)

    slot0 = slots[:, 0]
    slot1 = slots[:, 1]
    slot_flat = jnp.concatenate([slot0, slot1])
    tok_flat = jnp.concatenate(
        [jnp.arange(S, dtype=jnp.int32)] * 2)
    be = be_arr[:NBLK, 0]

    x_sorted = _sc_dispatch(h2, slot_flat, tok_flat)

    y_sorted = pl.pallas_call(
        _ffn_body,
        grid_spec=pltpu.PrefetchScalarGridSpec(
            num_scalar_prefetch=1,
            grid=(NBLK,),
            in_specs=[
                pl.BlockSpec((TB, D), lambda b, be_r: (b, 0)),
                pl.BlockSpec((1, D, F), lambda b, be_r: (be_r[b], 0, 0)),
                pl.BlockSpec((1, D, F), lambda b, be_r: (be_r[b], 0, 0)),
                pl.BlockSpec((1, F, D), lambda b, be_r: (be_r[b], 0, 0)),
            ],
            out_specs=pl.BlockSpec((TB, D), lambda b, be_r: (b, 0)),
        ),
        out_shape=jax.ShapeDtypeStruct((CAP, D), jnp.float32),
        compiler_params=pltpu.CompilerParams(
            vmem_limit_bytes=100 * 1024 * 1024),
    )(be, x_sorted, w_gate, w_up, w_down)

    y0, y1 = _sc_combine(y_sorted, slot0, slot1)

    out = pl.pallas_call(
        _final_body,
        grid=(nsb,),
        in_specs=[
            pl.BlockSpec((BQ, D), lambda i: (i, 0)),
            pl.BlockSpec((BQ, E), lambda i: (i, 0)),
            pl.BlockSpec((BQ, D), lambda i: (i, 0)),
            pl.BlockSpec((BQ, D), lambda i: (i, 0)),
        ],
        out_specs=pl.BlockSpec((BQ, D), lambda i: (i, 0)),
        out_shape=jax.ShapeDtypeStruct((S, D), jnp.float32),
    )(x2, wts, y0, y1)

    return out.reshape(B, S, D)
